# baseline (speedup 1.0000x reference)
import functools

import jax
import jax.numpy as jnp
from jax import lax
from jax.experimental import pallas as pl
from jax.experimental.pallas import tpu as pltpu, tpu_sc as plsc

D = 64
IDX_BLK = 128
KB = 2
B = KB * IDX_BLK
BP = B + 1
NC = 2
NS = 16
NW = NC * NS
L = 16
NG = 4


def _make_gather(n_batch: int, n_tok: int):
    nblk = n_batch // B
    ntasks = n_tok * nblk
    tasks_per_w = ntasks // NW
    nquads = tasks_per_w // NG

    mesh = plsc.VectorSubcoreMesh(core_axis_name="c", subcore_axis_name="s")

    @functools.partial(
        pl.kernel,
        mesh=mesh,
        out_type=jax.ShapeDtypeStruct((n_tok, D, n_batch), jnp.float32),
        scratch_types=[
            pltpu.VMEM((tasks_per_w * KB, IDX_BLK), jnp.int32),
            pltpu.VMEM((NG, B, D), jnp.float32),
            pltpu.VMEM((D, BP), jnp.float32),
            pltpu.VMEM((D, BP), jnp.float32),
            pltpu.SemaphoreType.DMA,
            pltpu.SemaphoreType.DMA,
            pltpu.SemaphoreType.DMA,
            pltpu.SemaphoreType.DMA,
            pltpu.SemaphoreType.DMA,
            pltpu.SemaphoreType.DMA,
        ],
        compiler_params=pltpu.CompilerParams(use_tc_tiling_on_sc=False,
                                             needs_layout_passes=False),
    )
    def gather_kernel(table_hbm, idx_hbm, out_hbm,
                      idx_v, rows_v, tr0, tr1,
                      g0, g1, g2, g3, w0, w1):
        wid = lax.axis_index("s") * NC + lax.axis_index("c")
        task0 = wid * tasks_per_w
        trows = (tr0, tr1)
        gsem = (g0, g1, g2, g3)
        wsem = (w0, w1)
        lane = lax.iota(jnp.int32, L)

        pltpu.sync_copy(idx_hbm.at[pl.ds(task0 * KB, tasks_per_w * KB)],
                        idx_v)

        def fire_g(i_local, rb):
            for j in range(KB):
                pltpu.async_copy(
                    table_hbm.at[idx_v.at[i_local * KB + j]],
                    rows_v.at[rb].at[pl.ds(j * IDX_BLK, IDX_BLK)],
                    gsem[rb],
                )

        def drain_g(rb):
            pltpu.make_async_copy(table_hbm.at[pl.ds(0, B)], rows_v.at[rb],
                                  gsem[rb]).wait()

        def transpose(rb, tb):
            src = rows_v.at[rb]
            dst = trows[tb]

            @plsc.parallel_loop(0, B, unroll=8)
            def _(r):
                rv = jnp.full((L,), r, jnp.int32)
                for cb in range(D // L):
                    v = src[r, pl.ds(cb * L, L)]
                    plsc.store_scatter(dst, [cb * L + lane, rv], v)

        def fire_w(i_local, tb):
            task = task0 + i_local
            t = task // nblk
            blk = task % nblk
            pltpu.async_copy(
                trows[tb].at[:, pl.ds(0, B)],
                out_hbm.at[t].at[:, pl.ds(blk * B, B)],
                wsem[tb],
            )

        def drain_w(tb):
            pltpu.make_async_copy(trows[tb].at[:, pl.ds(0, B)],
                                  out_hbm.at[0].at[:, pl.ds(0, B)],
                                  wsem[tb]).wait()

        for k in range(NG - 1):
            fire_g(k, k)

        def quad(q, carry):
            i = NG * q
            for k in range(NG):
                ik = i + k

                @pl.when(ik + NG - 1 <= tasks_per_w - 1)
                def _():
                    fire_g(ik + NG - 1, (k + NG - 1) % NG)
                drain_g(k)

                @pl.when(ik >= 2)
                def _():
                    drain_w(k % 2)
                transpose(k, k % 2)
                fire_w(ik, k % 2)
            return carry

        lax.fori_loop(0, nquads, quad, 0)
        drain_w(0)
        drain_w(1)

    return gather_kernel


def kernel(inputs, weight):
    nb, nt = inputs.shape
    idx_t = inputs.T.reshape(nt * nb // IDX_BLK, IDX_BLK).astype(jnp.int32)
    out_t = _make_gather(nb, nt)(weight, idx_t)
    return jnp.transpose(out_t, (2, 0, 1))


# speedup vs baseline: 1.0020x; 1.0020x over previous
import functools

import jax
import jax.numpy as jnp
from jax import lax
from jax.experimental import pallas as pl
from jax.experimental.pallas import tpu as pltpu, tpu_sc as plsc

D = 64
IDX_BLK = 128
KB = 2
B = KB * IDX_BLK
BP = B + 1
NC = 2
NS = 16
NW = NC * NS
L = 16
NG = 4


def _make_gather(n_batch: int, n_tok: int):
    nblk = n_batch // B
    ntasks = n_tok * nblk
    tasks_per_w = ntasks // NW
    nquads = tasks_per_w // NG

    mesh = plsc.VectorSubcoreMesh(core_axis_name="c", subcore_axis_name="s")

    @functools.partial(
        pl.kernel,
        mesh=mesh,
        out_type=jax.ShapeDtypeStruct((n_tok, D, n_batch), jnp.float32),
        scratch_types=[
            pltpu.VMEM((NG, B), jnp.int32),
            pltpu.VMEM((NG, B, D), jnp.float32),
            pltpu.VMEM((D, BP), jnp.float32),
            pltpu.VMEM((D, BP), jnp.float32),
            [pltpu.SemaphoreType.DMA] * NG,
            [pltpu.SemaphoreType.DMA] * NG,
            [pltpu.SemaphoreType.DMA] * 2,
        ],
        compiler_params=pltpu.CompilerParams(use_tc_tiling_on_sc=False,
                                             needs_layout_passes=False),
    )
    def gather_kernel(table_hbm, idx_hbm, out_hbm,
                      idx_v, rows_v, tr0, tr1, isem, gsem, wsem):
        wid = lax.axis_index("s") * NC + lax.axis_index("c")
        trows = (tr0, tr1)
        lane = lax.iota(jnp.int32, L)

        def task_id(i_local):
            return i_local * NW + wid

        def fire_i(i_local, kb):
            task = task_id(i_local)
            t = task // nblk
            blk = task % nblk
            pltpu.async_copy(idx_hbm.at[t, pl.ds(blk * B, B)],
                             idx_v.at[kb], isem[kb])

        def wait_i(kb):
            pltpu.make_async_copy(idx_hbm.at[0, pl.ds(0, B)], idx_v.at[kb],
                                  isem[kb]).wait()

        def fire_g(kb):
            for j in range(KB):
                pltpu.async_copy(
                    table_hbm.at[idx_v.at[kb].at[pl.ds(j * IDX_BLK,
                                                       IDX_BLK)]],
                    rows_v.at[kb].at[pl.ds(j * IDX_BLK, IDX_BLK)],
                    gsem[kb],
                )

        def drain_g(kb):
            pltpu.make_async_copy(table_hbm.at[pl.ds(0, B)], rows_v.at[kb],
                                  gsem[kb]).wait()

        def transpose(kb, tb):
            src = rows_v.at[kb]
            dst = trows[tb]

            @plsc.parallel_loop(0, B, unroll=8)
            def _(r):
                rv = jnp.full((L,), r, jnp.int32)
                for cb in range(D // L):
                    v = src[r, pl.ds(cb * L, L)]
                    plsc.store_scatter(dst, [cb * L + lane, rv], v)

        def fire_w(i_local, tb):
            task = task_id(i_local)
            t = task // nblk
            blk = task % nblk
            pltpu.async_copy(
                trows[tb].at[:, pl.ds(0, B)],
                out_hbm.at[t].at[:, pl.ds(blk * B, B)],
                wsem[tb],
            )

        def drain_w(tb):
            pltpu.make_async_copy(trows[tb].at[:, pl.ds(0, B)],
                                  out_hbm.at[0].at[:, pl.ds(0, B)],
                                  wsem[tb]).wait()

        for k in range(NG):
            fire_i(k, k)
        for k in range(NG - 1):
            wait_i(k)
            fire_g(k)

        def quad(q, carry):
            i = NG * q
            for k in range(NG):
                ik = i + k

                @pl.when(ik + NG - 1 <= tasks_per_w - 1)
                def _():
                    wait_i((k + NG - 1) % NG)
                    fire_g((k + NG - 1) % NG)

                drain_g(k)

                @pl.when(ik + NG <= tasks_per_w - 1)
                def _():
                    fire_i(ik + NG, k)

                @pl.when(ik >= 2)
                def _():
                    drain_w(k % 2)
                transpose(k, k % 2)
                fire_w(ik, k % 2)
            return carry

        lax.fori_loop(0, nquads, quad, 0)
        drain_w(0)
        drain_w(1)

    return gather_kernel


def kernel(inputs, weight):
    nb, nt = inputs.shape
    idx_t = inputs.T.astype(jnp.int32)
    out_t = _make_gather(nb, nt)(weight, idx_t)
    return jnp.transpose(out_t, (2, 0, 1))


# speedup vs baseline: 1.0031x; 1.0011x over previous
import functools

import jax
import jax.numpy as jnp
from jax import lax
from jax.experimental import pallas as pl
from jax.experimental.pallas import tpu as pltpu, tpu_sc as plsc

D = 64
IDX_BLK = 128
KB = 2
B = KB * IDX_BLK
BP = B + 1
NC = 2
NS = 16
NW = NC * NS
L = 16
NG = 4


def _make_gather(n_batch: int, n_tok: int):
    nblk = n_batch // B
    ntasks = n_tok * nblk
    tasks_per_w = ntasks // NW
    nquads = tasks_per_w // NG

    mesh = plsc.VectorSubcoreMesh(core_axis_name="c", subcore_axis_name="s")

    @functools.partial(
        pl.kernel,
        mesh=mesh,
        out_type=jax.ShapeDtypeStruct((n_tok, D, n_batch), jnp.float32),
        scratch_types=[
            pltpu.VMEM((NG, B), jnp.int32),
            pltpu.VMEM((NG, B, D), jnp.float32),
            pltpu.VMEM((D, BP), jnp.float32),
            pltpu.VMEM((D, BP), jnp.float32),
            [pltpu.SemaphoreType.DMA] * NG,
            [pltpu.SemaphoreType.DMA] * NG,
            [pltpu.SemaphoreType.DMA] * 2,
        ],
        compiler_params=pltpu.CompilerParams(use_tc_tiling_on_sc=False,
                                             needs_layout_passes=False),
    )
    def gather_kernel(table_hbm, idx_hbm, out_hbm,
                      idx_v, rows_v, tr0, tr1, isem, gsem, wsem):
        wid = lax.axis_index("s") * NC + lax.axis_index("c")
        trows = (tr0, tr1)
        lane = lax.iota(jnp.int32, L)

        def task_id(i_local):
            return i_local * NW + wid

        def fire_i(i_local, kb):
            task = task_id(i_local)
            t = task // nblk
            blk = task % nblk
            pltpu.async_copy(idx_hbm.at[t, pl.ds(blk * B, B)],
                             idx_v.at[kb], isem[kb])

        def wait_i(kb):
            pltpu.make_async_copy(idx_hbm.at[0, pl.ds(0, B)], idx_v.at[kb],
                                  isem[kb]).wait()

        def fire_g(kb):
            for j in range(KB):
                pltpu.async_copy(
                    table_hbm.at[idx_v.at[kb].at[pl.ds(j * IDX_BLK,
                                                       IDX_BLK)]],
                    rows_v.at[kb].at[pl.ds(j * IDX_BLK, IDX_BLK)],
                    gsem[kb],
                )

        def drain_g(kb):
            pltpu.make_async_copy(table_hbm.at[pl.ds(0, B)], rows_v.at[kb],
                                  gsem[kb]).wait()

        def transpose(kb, tb):
            src = rows_v.at[kb]
            dst = trows[tb]

            @plsc.parallel_loop(0, B, unroll=8)
            def _(r):
                rv = jnp.full((L,), r, jnp.int32)
                for cb in range(D // L):
                    v = src[r, pl.ds(cb * L, L)]
                    plsc.store_scatter(dst, [cb * L + lane, rv], v)

        def fire_w(i_local, tb):
            task = task_id(i_local)
            t = task // nblk
            blk = task % nblk
            pltpu.async_copy(
                trows[tb].at[:, pl.ds(0, B)],
                out_hbm.at[t].at[:, pl.ds(blk * B, B)],
                wsem[tb],
            )

        def drain_w(tb):
            pltpu.make_async_copy(trows[tb].at[:, pl.ds(0, B)],
                                  out_hbm.at[0].at[:, pl.ds(0, B)],
                                  wsem[tb]).wait()

        for k in range(NG):
            fire_i(k, k)
        for k in range(NG - 1):
            wait_i(k)
            fire_g(k)

        def quad(q, carry):
            i = NG * q
            for k in range(NG):
                ik = i + k

                @pl.when(ik + NG - 1 <= tasks_per_w - 1)
                def _():
                    wait_i((k + NG - 1) % NG)
                    fire_g((k + NG - 1) % NG)

                drain_g(k)

                @pl.when(ik + NG <= tasks_per_w - 1)
                def _():
                    fire_i(ik + NG, k)

                @pl.when(ik >= 2)
                def _():
                    drain_w(k % 2)
                transpose(k, k % 2)
                fire_w(ik, k % 2)
            return carry

        lax.fori_loop(0, nquads, quad, 0)
        drain_w(0)
        drain_w(1)

    return gather_kernel


def kernel(inputs, weight):
    nb, nt = inputs.shape
    idx_t = inputs.T.astype(jnp.int32)
    w2 = lax.optimization_barrier(weight.reshape(weight.shape[0] // 2,
                                                 2 * D))
    out_t = _make_gather(nb, nt)(w2.reshape(weight.shape), idx_t)
    return jnp.transpose(out_t, (2, 0, 1))
